# baseline (speedup 1.0000x reference)
import functools

import jax
import jax.numpy as jnp
from jax import lax
from jax.experimental import pallas as pl
from jax.experimental.pallas import tpu as pltpu
from jax.experimental.pallas import tpu_sc as plsc

_N = 50000
_C = 128
_FE = 9
_G = 32
_EPS = 1e-5

_NC = 2
_NS = 16
_NW = _NC * _NS
_VS = 14
_SPI = 2
_VPI = _VS * _SPI
_ITERS = 30
_PW = _VPI * _ITERS
_NP = _PW * _NW
_BV = 512
_MT = 46 * _BV


def _stats_call(lv):
    rows = 5000
    grid = _N // rows

    def kern(x_ref, o_ref, acc_ref):
        i = pl.program_id(0)

        @pl.when(i == 0)
        def _init():
            acc_ref[...] = jnp.zeros_like(acc_ref)

        x = x_ref[...]
        acc_ref[0:1, :] = acc_ref[0:1, :] + jnp.sum(x, axis=0, keepdims=True)
        acc_ref[1:2, :] = acc_ref[1:2, :] + jnp.sum(x * x, axis=0, keepdims=True)

        @pl.when(i == grid - 1)
        def _out():
            o_ref[...] = acc_ref[...]

    return pl.pallas_call(
        kern,
        grid=(grid,),
        in_specs=[pl.BlockSpec((rows, _C), lambda i: (i, 0))],
        out_specs=pl.BlockSpec((2, _C), lambda i: (0, 0)),
        out_shape=jax.ShapeDtypeStruct((2, _C), jnp.float32),
        scratch_shapes=[pltpu.VMEM((2, _C), jnp.float32)],
    )(lv)


def _sc_conv(lv, idx_p, params):
    mesh = plsc.VectorSubcoreMesh(core_axis_name="c", subcore_axis_name="s")

    @functools.partial(
        pl.kernel,
        mesh=mesh,
        out_type=jax.ShapeDtypeStruct((_NP, _C), jnp.float32),
        scratch_types=[
            pltpu.VMEM((_ITERS, _SPI, 128), jnp.int32),
            pltpu.VMEM((2, _SPI * 128, _C), jnp.float32),
            pltpu.VMEM((12, _C), jnp.float32),
            pltpu.VMEM((2, 2 * _VPI, _C), jnp.float32),
            pltpu.SemaphoreType.DMA,
            pltpu.SemaphoreType.DMA,
            pltpu.SemaphoreType.DMA,
            pltpu.SemaphoreType.DMA,
        ],
    )
    def k(lv_hbm, idx_hbm, par_hbm, out_hbm, idx_v, rows_v, par_v, out_v,
          sem_g0, sem_g1, sem_o0, sem_o1):
        wid = lax.axis_index("s") * _NC + lax.axis_index("c")
        base = wid * _PW
        sems = (sem_g0, sem_g1)
        pltpu.sync_copy(par_hbm, par_v)
        pltpu.sync_copy(idx_hbm.at[wid], idx_v)

        def fire_gathers(step, buf, sem):
            for s in range(_SPI):
                pltpu.async_copy(
                    lv_hbm.at[idx_v.at[step, s]],
                    rows_v.at[buf, pl.ds(s * 128, 128)],
                    sem,
                )

        def wait_gathers(buf, sem):
            pltpu.make_async_copy(
                lv_hbm.at[pl.ds(0, _SPI * 128)], rows_v.at[buf], sem
            ).wait()

        def out_slice(k_step):
            off = pl.multiple_of(base + (k_step - 1) * _VPI, 2 * _VPI)
            return out_hbm.at[pl.ds(off, 2 * _VPI)]

        def wait_out(sem):
            pltpu.make_async_copy(out_v.at[0], out_slice(1), sem).wait()

        def compute(k_step):
            kbd = k_step % 2
            pbd = (k_step // 2) % 2
            for ch in range(_C // 16):
                sl = pl.ds(ch * 16, 16)
                wv = [par_v[t, sl] for t in range(_FE)]
                sc = par_v[9, sl]
                of = par_v[10, sl]
                bi = par_v[11, sl]
                for s in range(_SPI):

                    def vbody(u, c, s=s, sl=sl, wv=wv, sc=sc, of=of, bi=bi):
                        rbase = s * 128 + u * _FE
                        acc = bi
                        for t in range(_FE):
                            r = rows_v[kbd, rbase + t, sl]
                            xr = jnp.maximum(r * sc + of, 0.0)
                            acc = acc + xr * wv[t]
                        out_v[pbd, kbd * _VPI + s * _VS + u, sl] = acc
                        return c

                    lax.fori_loop(0, _VS, vbody, 0)

        fire_gathers(0, 0, sem_g0)

        def body(k_step, carry):
            kb = k_step % 2
            for cur in range(2):
                nxt = 1 - cur

                @pl.when(kb == cur)
                def _(cur=cur, nxt=nxt):
                    @pl.when(k_step < _ITERS - 1)
                    def _():
                        fire_gathers(k_step + 1, nxt, sems[nxt])

                    wait_gathers(cur, sems[cur])

            @pl.when((k_step % 4 == 0) & (k_step >= 4))
            def _():
                wait_out(sem_o0)

            @pl.when((k_step % 4 == 2) & (k_step >= 6))
            def _():
                wait_out(sem_o1)

            compute(k_step)

            @pl.when(k_step % 4 == 1)
            def _():
                pltpu.async_copy(out_v.at[0], out_slice(k_step), sem_o0)

            @pl.when(k_step % 4 == 3)
            def _():
                pltpu.async_copy(out_v.at[1], out_slice(k_step), sem_o1)

            return carry

        lax.fori_loop(0, _ITERS, body, 0)
        wait_out(sem_o0)
        wait_out(sem_o1)

    return k(lv, idx_p, params)


def _tc_gather_call(lv, par, idx_tc):
    grid = _MT // _BV

    def kern(idx_ref, lv_ref, p_ref, o_ref):
        sc = p_ref[9:10, :]
        of = p_ref[10:11, :]

        def vb(g, c):
            base = pl.multiple_of(g * 16, 16)
            acc = jnp.broadcast_to(p_ref[11:12, :], (16, _C))
            for t in range(_FE):
                rows = [lv_ref[pl.ds(idx_ref[base + j, t], 1), :]
                        for j in range(16)]
                r16 = jnp.concatenate(rows, axis=0)
                acc = acc + jnp.maximum(r16 * sc + of, 0.0) * p_ref[t:t + 1, :]
            o_ref[pl.ds(base, 16), :] = acc
            return c

        lax.fori_loop(0, _BV // 16, vb, 0)

    return pl.pallas_call(
        kern,
        grid=(grid,),
        in_specs=[
            pl.BlockSpec((_BV, _FE), lambda i: (i, 0),
                         memory_space=pltpu.SMEM),
            pl.BlockSpec((_N, _C), lambda i: (0, 0)),
            pl.BlockSpec((12, _C), lambda i: (0, 0)),
        ],
        out_specs=pl.BlockSpec((_BV, _C), lambda i: (i, 0)),
        out_shape=jax.ShapeDtypeStruct((_MT, _C), jnp.float32),
    )(idx_tc, lv, par)


def kernel(lv, gamma, beta, weight, bias, neighbor_idx):
    stats = _stats_call(lv)
    cnt = jnp.float32(_N * (_C // _G))
    s = stats[0].reshape(_G, _C // _G).sum(axis=1)
    ss = stats[1].reshape(_G, _C // _G).sum(axis=1)
    mean = s / cnt
    var = jnp.maximum(ss / cnt - mean * mean, 0.0)
    inv = lax.rsqrt(var + _EPS)
    rep = _C // _G
    scale_c = gamma * jnp.repeat(inv, rep)
    offset_c = beta - jnp.repeat(mean * inv, rep) * gamma
    params = jnp.concatenate(
        [weight, scale_c[None], offset_c[None], bias[None]], axis=0)

    idx = neighbor_idx.astype(jnp.int32)
    idx_p = idx[:_NP].reshape(_NW, _ITERS, _SPI, _VS * _FE)
    idx_p = jnp.pad(idx_p, ((0, 0), (0, 0), (0, 0), (0, 2)))
    idx_tc = jnp.pad(idx[_NP:], ((0, _MT - (_N - _NP)), (0, 0)))

    out_sc = _sc_conv(lv, idx_p, params)
    out_tc = _tc_gather_call(lv, params, idx_tc)
    return jnp.concatenate([out_sc, out_tc[:_N - _NP]], axis=0)


# speedup vs baseline: 1.0547x; 1.0547x over previous
import functools

import jax
import jax.numpy as jnp
from jax import lax
from jax.experimental import pallas as pl
from jax.experimental.pallas import tpu as pltpu
from jax.experimental.pallas import tpu_sc as plsc

_N = 50000
_C = 128
_FE = 9
_G = 32
_EPS = 1e-5

_NC = 2
_NS = 16
_NW = _NC * _NS
_VS = 14
_SPI = 2
_VPI = _VS * _SPI
_ITERS = 32
_PW = _VPI * _ITERS
_NP = _PW * _NW
_BV = 512
_MT = 42 * _BV


def _stats_call(lv):
    rows = 5000
    grid = _N // rows

    def kern(x_ref, o_ref, acc_ref):
        i = pl.program_id(0)

        @pl.when(i == 0)
        def _init():
            acc_ref[...] = jnp.zeros_like(acc_ref)

        x = x_ref[...]
        acc_ref[0:1, :] = acc_ref[0:1, :] + jnp.sum(x, axis=0, keepdims=True)
        acc_ref[1:2, :] = acc_ref[1:2, :] + jnp.sum(x * x, axis=0, keepdims=True)

        @pl.when(i == grid - 1)
        def _out():
            o_ref[...] = acc_ref[...]

    return pl.pallas_call(
        kern,
        grid=(grid,),
        in_specs=[pl.BlockSpec((rows, _C), lambda i: (i, 0))],
        out_specs=pl.BlockSpec((2, _C), lambda i: (0, 0)),
        out_shape=jax.ShapeDtypeStruct((2, _C), jnp.float32),
        scratch_shapes=[pltpu.VMEM((2, _C), jnp.float32)],
    )(lv)


def _sc_conv(lv, idx_p, params):
    mesh = plsc.VectorSubcoreMesh(core_axis_name="c", subcore_axis_name="s")

    @functools.partial(
        pl.kernel,
        mesh=mesh,
        out_type=jax.ShapeDtypeStruct((_NP, _C), jnp.float32),
        scratch_types=[
            pltpu.VMEM((_ITERS, _SPI, 128), jnp.int32),
            pltpu.VMEM((2, _SPI * 128, _C), jnp.float32),
            pltpu.VMEM((12, _C), jnp.float32),
            pltpu.VMEM((2, 2 * _VPI, _C), jnp.float32),
            pltpu.SemaphoreType.DMA,
            pltpu.SemaphoreType.DMA,
            pltpu.SemaphoreType.DMA,
            pltpu.SemaphoreType.DMA,
        ],
    )
    def k(lv_hbm, idx_hbm, par_hbm, out_hbm, idx_v, rows_v, par_v, out_v,
          sem_g0, sem_g1, sem_o0, sem_o1):
        wid = lax.axis_index("s") * _NC + lax.axis_index("c")
        base = wid * _PW
        sems = (sem_g0, sem_g1)
        pltpu.sync_copy(par_hbm, par_v)
        pltpu.sync_copy(idx_hbm.at[wid], idx_v)

        def fire_gathers(step, buf, sem):
            for s in range(_SPI):
                pltpu.async_copy(
                    lv_hbm.at[idx_v.at[step, s]],
                    rows_v.at[buf, pl.ds(s * 128, 128)],
                    sem,
                )

        def wait_gathers(buf, sem):
            pltpu.make_async_copy(
                lv_hbm.at[pl.ds(0, _SPI * 128)], rows_v.at[buf], sem
            ).wait()

        def out_slice(k_step):
            off = pl.multiple_of(base + (k_step - 1) * _VPI, 2 * _VPI)
            return out_hbm.at[pl.ds(off, 2 * _VPI)]

        def wait_out(sem):
            pltpu.make_async_copy(out_v.at[0], out_slice(1), sem).wait()

        def compute(k_step):
            kbd = k_step % 2
            pbd = (k_step // 2) % 2
            for ch in range(_C // 16):
                sl = pl.ds(ch * 16, 16)
                wv = [par_v[t, sl] for t in range(_FE)]
                sc = par_v[9, sl]
                of = par_v[10, sl]
                bi = par_v[11, sl]
                for s in range(_SPI):

                    def vbody(u, c, s=s, sl=sl, wv=wv, sc=sc, of=of, bi=bi):
                        rbase = s * 128 + u * _FE
                        acc = bi
                        for t in range(_FE):
                            r = rows_v[kbd, rbase + t, sl]
                            xr = jnp.maximum(r * sc + of, 0.0)
                            acc = acc + xr * wv[t]
                        out_v[pbd, kbd * _VPI + s * _VS + u, sl] = acc
                        return c

                    lax.fori_loop(0, _VS, vbody, 0)

        fire_gathers(0, 0, sem_g0)

        def body(k_step, carry):
            kb = k_step % 2
            for cur in range(2):
                nxt = 1 - cur

                @pl.when(kb == cur)
                def _(cur=cur, nxt=nxt):
                    @pl.when(k_step < _ITERS - 1)
                    def _():
                        fire_gathers(k_step + 1, nxt, sems[nxt])

                    wait_gathers(cur, sems[cur])

            @pl.when((k_step % 4 == 0) & (k_step >= 4))
            def _():
                wait_out(sem_o0)

            @pl.when((k_step % 4 == 2) & (k_step >= 6))
            def _():
                wait_out(sem_o1)

            compute(k_step)

            @pl.when(k_step % 4 == 1)
            def _():
                pltpu.async_copy(out_v.at[0], out_slice(k_step), sem_o0)

            @pl.when(k_step % 4 == 3)
            def _():
                pltpu.async_copy(out_v.at[1], out_slice(k_step), sem_o1)

            return carry

        lax.fori_loop(0, _ITERS, body, 0)
        wait_out(sem_o0)
        wait_out(sem_o1)

    return k(lv, idx_p, params)


def _tc_gather_call(lv, par, idx_tc):
    grid = _MT // _BV

    def kern(idx_ref, lv_ref, p_ref, o_ref):
        sc = p_ref[9:10, :]
        of = p_ref[10:11, :]

        def vb(g, c):
            base = pl.multiple_of(g * 8, 8)
            acc = jnp.broadcast_to(p_ref[11:12, :], (8, _C))
            for t in range(_FE):
                rows = [lv_ref[pl.ds(idx_ref[base + j, t], 1), :]
                        for j in range(8)]
                r8 = jnp.concatenate(rows, axis=0)
                acc = acc + jnp.maximum(r8 * sc + of, 0.0) * p_ref[t:t + 1, :]
            o_ref[pl.ds(base, 8), :] = acc
            return c

        lax.fori_loop(0, _BV // 8, vb, 0)

    return pl.pallas_call(
        kern,
        grid=(grid,),
        in_specs=[
            pl.BlockSpec((_BV, _FE), lambda i: (i, 0),
                         memory_space=pltpu.SMEM),
            pl.BlockSpec((_N, _C), lambda i: (0, 0)),
            pl.BlockSpec((12, _C), lambda i: (0, 0)),
        ],
        out_specs=pl.BlockSpec((_BV, _C), lambda i: (i, 0)),
        out_shape=jax.ShapeDtypeStruct((_MT, _C), jnp.float32),
    )(idx_tc, lv, par)


def kernel(lv, gamma, beta, weight, bias, neighbor_idx):
    stats = _stats_call(lv)
    cnt = jnp.float32(_N * (_C // _G))
    s = stats[0].reshape(_G, _C // _G).sum(axis=1)
    ss = stats[1].reshape(_G, _C // _G).sum(axis=1)
    mean = s / cnt
    var = jnp.maximum(ss / cnt - mean * mean, 0.0)
    inv = lax.rsqrt(var + _EPS)
    rep = _C // _G
    scale_c = gamma * jnp.repeat(inv, rep)
    offset_c = beta - jnp.repeat(mean * inv, rep) * gamma
    params = jnp.concatenate(
        [weight, scale_c[None], offset_c[None], bias[None]], axis=0)

    idx = neighbor_idx.astype(jnp.int32)
    idx_p = idx[:_NP].reshape(_NW, _ITERS, _SPI, _VS * _FE)
    idx_p = jnp.pad(idx_p, ((0, 0), (0, 0), (0, 0), (0, 2)))
    idx_tc = jnp.pad(idx[_NP:], ((0, _MT - (_N - _NP)), (0, 0)))

    out_sc = _sc_conv(lv, idx_p, params)
    out_tc = _tc_gather_call(lv, params, idx_tc)
    return jnp.concatenate([out_sc, out_tc[:_N - _NP]], axis=0)
